# Initial kernel scaffold; baseline (speedup 1.0000x reference)
import jax
import jax.numpy as jnp
from jax.experimental import pallas as pl


def kernel(state, action_table):
    raise NotImplementedError("write your pallas kernel here")


# speedup vs baseline: 3.4036x; 3.4036x over previous
import jax
import jax.numpy as jnp
from jax.experimental import pallas as pl
from jax.experimental.pallas import tpu as pltpu

_NUM_METRIC = 10
_NUM_TASK = 8
_NUM_ACTIONS = 1024
_ROW_BLOCK = 512


def _onehot_body(x_ref, tbl_ref, probs_ref, logits_ref):
    r = x_ref.shape[0]
    x = x_ref[...]
    metric = x[:, :_NUM_METRIC]
    task = x[:, _NUM_METRIC:_NUM_METRIC + _NUM_TASK]

    mcol = jax.lax.broadcasted_iota(jnp.int32, (r, _NUM_METRIC), 1)
    mmax = jnp.max(metric, axis=1, keepdims=True)
    midx = jnp.min(jnp.where(metric == mmax, mcol, _NUM_METRIC), axis=1,
                   keepdims=True)
    tcol = jax.lax.broadcasted_iota(jnp.int32, (r, _NUM_TASK), 1)
    tmax = jnp.max(task, axis=1, keepdims=True)
    tidx = jnp.min(jnp.where(task == tmax, tcol, _NUM_TASK), axis=1,
                   keepdims=True)

    flat = tidx * _NUM_METRIC + midx

    tbl = tbl_ref[...]
    k = jax.lax.broadcasted_iota(jnp.int32, (r, _NUM_TASK * _NUM_METRIC), 1)
    aidx = jnp.sum(jnp.where(k == flat, jnp.broadcast_to(tbl, k.shape), 0),
                   axis=1, keepdims=True)

    cols = jax.lax.broadcasted_iota(jnp.int32, (r, _NUM_ACTIONS), 1)
    hit = cols == aidx
    probs_ref[...] = jnp.where(hit, jnp.float32(1.0), jnp.float32(0.0))
    logits_ref[...] = jnp.where(hit, jnp.float32(0.0), jnp.float32(-1000000.0))


def kernel(state, action_table):
    s = state.astype(jnp.float32)
    b = s.shape[0]
    x = s[:, 1:1 + _NUM_METRIC + _NUM_TASK]
    tbl = action_table.reshape(1, _NUM_TASK * _NUM_METRIC).astype(jnp.int32)

    grid = (b // _ROW_BLOCK,)
    probs, logits = pl.pallas_call(
        _onehot_body,
        grid=grid,
        in_specs=[
            pl.BlockSpec((_ROW_BLOCK, _NUM_METRIC + _NUM_TASK),
                         lambda i: (i, 0)),
            pl.BlockSpec((1, _NUM_TASK * _NUM_METRIC), lambda i: (0, 0)),
        ],
        out_specs=[
            pl.BlockSpec((_ROW_BLOCK, _NUM_ACTIONS), lambda i: (i, 0)),
            pl.BlockSpec((_ROW_BLOCK, _NUM_ACTIONS), lambda i: (i, 0)),
        ],
        out_shape=[
            jax.ShapeDtypeStruct((b, _NUM_ACTIONS), jnp.float32),
            jax.ShapeDtypeStruct((b, _NUM_ACTIONS), jnp.float32),
        ],
        compiler_params=pltpu.CompilerParams(
            dimension_semantics=("parallel",)),
    )(x, tbl)

    feature_vector = jnp.zeros((b, 1), dtype=jnp.float32)
    return (probs, logits, probs, feature_vector)
